# Initial kernel scaffold; baseline (speedup 1.0000x reference)
import jax
import jax.numpy as jnp
from jax.experimental import pallas as pl


def kernel(inputs):
    raise NotImplementedError("write your pallas kernel here")


# speedup vs baseline: 35.0748x; 35.0748x over previous
import functools

import jax
import jax.numpy as jnp
from jax import lax
from jax.experimental import pallas as pl
from jax.experimental.pallas import tpu as pltpu
from jax.experimental.pallas import tpu_sc as plsc

K = 8
B = 16
S = 8192
C = 128
NC = 2
LANES = 16
NG = 4
CH_HALF = NG * LANES
CHUNK = 512
NCHUNK = S // CHUNK

_mesh = plsc.VectorSubcoreMesh(core_axis_name="c", subcore_axis_name="s")


@functools.partial(
    pl.kernel,
    out_type=jax.ShapeDtypeStruct((B, C * K), jnp.float32),
    mesh=_mesh,
    scratch_types=[
        pltpu.VMEM((CHUNK, CH_HALF), jnp.float32),
        pltpu.VMEM((CHUNK, CH_HALF), jnp.float32),
        pltpu.VMEM((CH_HALF * K,), jnp.float32),
        pltpu.SemaphoreType.DMA,
        pltpu.SemaphoreType.DMA,
    ],
    compiler_params=pltpu.CompilerParams(
        use_tc_tiling_on_sc=False, needs_layout_passes=False
    ),
)
def _topk_sc(x_hbm, out_hbm, buf0, buf1, obuf, sem0, sem1):
    wid = lax.axis_index("s") * NC + lax.axis_index("c")
    b = wid // 2
    ch0 = (wid % 2) * CH_HALF

    neg = jnp.full((LANES,), -jnp.inf, dtype=jnp.float32)
    states = tuple(tuple(neg for _ in range(K)) for _ in range(NG))

    bufs = (buf0, buf1)
    sems = (sem0, sem1)
    copies = [None, None]

    def start(i):
        copies[i % 2] = pltpu.async_copy(
            x_hbm.at[b, pl.ds(i * CHUNK, CHUNK), pl.ds(ch0, CH_HALF)],
            bufs[i % 2],
            sems[i % 2],
        )

    start(0)
    for chunk in range(NCHUNK):
        copies[chunk % 2].wait()
        if chunk + 1 < NCHUNK:
            start(chunk + 1)
        buf = bufs[chunk % 2]

        def body(s, st, buf=buf):
            out_st = []
            for g in range(NG):
                v = buf[s, pl.ds(g * LANES, LANES)]
                ts = list(st[g])
                for j in range(K):
                    hi = jnp.maximum(ts[j], v)
                    v = jnp.minimum(ts[j], v)
                    ts[j] = hi
                out_st.append(tuple(ts))
            return tuple(out_st)

        states = lax.fori_loop(0, CHUNK, body, states)

    lane = lax.iota(jnp.int32, LANES)
    for g in range(NG):
        for j in range(K):
            idx = lane * K + (g * LANES * K + j)
            plsc.store_scatter(obuf, [idx], states[g][j])
    pltpu.sync_copy(obuf, out_hbm.at[b, pl.ds(ch0 * K, CH_HALF * K)])


def kernel(inputs):
    x = inputs.reshape(B, S, C)
    return _topk_sc(x)


# speedup vs baseline: 43.7402x; 1.2471x over previous
import functools

import jax
import jax.numpy as jnp
from jax import lax
from jax.experimental import pallas as pl
from jax.experimental.pallas import tpu as pltpu
from jax.experimental.pallas import tpu_sc as plsc

K = 8
B = 16
S = 8192
C = 128
NC = 2
LANES = 16
NG = 4
CH_HALF = NG * LANES
CHUNK = 512
NCHUNK = S // CHUNK

WIN = 8
NWIN = CHUNK // WIN

_NET8 = (
    (0, 1), (2, 3), (4, 5), (6, 7),
    (0, 2), (1, 3), (4, 6), (5, 7),
    (1, 2), (5, 6), (0, 4), (3, 7),
    (1, 5), (2, 6),
    (1, 4), (3, 6),
    (2, 4), (3, 5),
    (3, 4),
)
_BITONIC8 = (
    (0, 4), (1, 5), (2, 6), (3, 7),
    (0, 2), (1, 3), (4, 6), (5, 7),
    (0, 1), (2, 3), (4, 5), (6, 7),
)


def _ce(b, i, j):
    hi = jnp.maximum(b[i], b[j])
    lo = jnp.minimum(b[i], b[j])
    b[i] = hi
    b[j] = lo


_mesh = plsc.VectorSubcoreMesh(core_axis_name="c", subcore_axis_name="s")


@functools.partial(
    pl.kernel,
    out_type=jax.ShapeDtypeStruct((B, C * K), jnp.float32),
    mesh=_mesh,
    scratch_types=[
        pltpu.VMEM((CHUNK, CH_HALF), jnp.float32),
        pltpu.VMEM((CHUNK, CH_HALF), jnp.float32),
        pltpu.VMEM((CH_HALF * K,), jnp.float32),
        pltpu.SemaphoreType.DMA,
        pltpu.SemaphoreType.DMA,
    ],
    compiler_params=pltpu.CompilerParams(
        use_tc_tiling_on_sc=False, needs_layout_passes=False
    ),
)
def _topk_sc(x_hbm, out_hbm, buf0, buf1, obuf, sem0, sem1):
    wid = lax.axis_index("s") * NC + lax.axis_index("c")
    b = wid // 2
    ch0 = (wid % 2) * CH_HALF

    neg = jnp.full((LANES,), -jnp.inf, dtype=jnp.float32)
    states = tuple(tuple(neg for _ in range(K)) for _ in range(NG))

    bufs = (buf0, buf1)
    sems = (sem0, sem1)
    copies = [None, None]

    def start(i):
        copies[i % 2] = pltpu.async_copy(
            x_hbm.at[b, pl.ds(i * CHUNK, CHUNK), pl.ds(ch0, CH_HALF)],
            bufs[i % 2],
            sems[i % 2],
        )

    start(0)
    for chunk in range(NCHUNK):
        copies[chunk % 2].wait()
        if chunk + 1 < NCHUNK:
            start(chunk + 1)
        buf = bufs[chunk % 2]

        def body(w, st, buf=buf):
            out_st = []
            for g in range(NG):
                wb = [buf[w * WIN + r, pl.ds(g * LANES, LANES)] for r in range(WIN)]
                for (i, j) in _NET8:
                    _ce(wb, i, j)
                ts = [jnp.maximum(st[g][i], wb[K - 1 - i]) for i in range(K)]
                for (i, j) in _BITONIC8:
                    _ce(ts, i, j)
                out_st.append(tuple(ts))
            return tuple(out_st)

        states = lax.fori_loop(0, NWIN, body, states)

    lane = lax.iota(jnp.int32, LANES)
    for g in range(NG):
        for j in range(K):
            idx = lane * K + (g * LANES * K + j)
            plsc.store_scatter(obuf, [idx], states[g][j])
    pltpu.sync_copy(obuf, out_hbm.at[b, pl.ds(ch0 * K, CH_HALF * K)])


def kernel(inputs):
    x = inputs.reshape(B, S, C)
    return _topk_sc(x)


# speedup vs baseline: 54.2921x; 1.2412x over previous
import functools

import jax
import jax.numpy as jnp
from jax import lax
from jax.experimental import pallas as pl
from jax.experimental.pallas import tpu as pltpu
from jax.experimental.pallas import tpu_sc as plsc

K = 8
B = 16
S = 8192
C = 128
NC = 2
LANES = 16
NG = 4
CH_HALF = NG * LANES
CHUNK = 512
NCHUNK = S // CHUNK

WIN = 8
NWIN = CHUNK // WIN

_NET8 = (
    (0, 1), (2, 3), (4, 5), (6, 7),
    (0, 2), (1, 3), (4, 6), (5, 7),
    (1, 2), (5, 6), (0, 4), (3, 7),
    (1, 5), (2, 6),
    (1, 4), (3, 6),
    (2, 4), (3, 5),
    (3, 4),
)
_BITONIC8 = (
    (0, 4), (1, 5), (2, 6), (3, 7),
    (0, 2), (1, 3), (4, 6), (5, 7),
    (0, 1), (2, 3), (4, 5), (6, 7),
)


def _ce(b, i, j):
    hi = jnp.maximum(b[i], b[j])
    lo = jnp.minimum(b[i], b[j])
    b[i] = hi
    b[j] = lo


_mesh = plsc.VectorSubcoreMesh(core_axis_name="c", subcore_axis_name="s")


@functools.partial(
    pl.kernel,
    out_type=jax.ShapeDtypeStruct((B, C * K), jnp.float32),
    mesh=_mesh,
    scratch_types=[
        pltpu.VMEM((CHUNK, CH_HALF), jnp.float32),
        pltpu.VMEM((CHUNK, CH_HALF), jnp.float32),
        pltpu.VMEM((CH_HALF * K,), jnp.float32),
        pltpu.SemaphoreType.DMA,
        pltpu.SemaphoreType.DMA,
    ],
    compiler_params=pltpu.CompilerParams(
        use_tc_tiling_on_sc=False, needs_layout_passes=False
    ),
)
def _topk_sc(x_hbm, out_hbm, buf0, buf1, obuf, sem0, sem1):
    wid = lax.axis_index("s") * NC + lax.axis_index("c")
    b = wid // 2
    ch0 = (wid % 2) * CH_HALF

    neg = jnp.full((LANES,), -jnp.inf, dtype=jnp.float32)
    states = tuple(tuple(neg for _ in range(K)) for _ in range(NG))

    bufs = (buf0, buf1)
    sems = (sem0, sem1)
    copies = [None, None]

    def start(i):
        copies[i % 2] = pltpu.async_copy(
            x_hbm.at[b, pl.ds(i * CHUNK, CHUNK), pl.ds(ch0, CH_HALF)],
            bufs[i % 2],
            sems[i % 2],
        )

    start(0)
    for chunk in range(NCHUNK):
        copies[chunk % 2].wait()
        if chunk + 1 < NCHUNK:
            start(chunk + 1)
        buf = bufs[chunk % 2]

        new_states = []
        for half in range(NG // 2):
            def body(w, st, buf=buf, half=half):
                out_st = []
                for gg in range(2):
                    g = half * 2 + gg
                    wb = [
                        buf[w * WIN + r, pl.ds(g * LANES, LANES)]
                        for r in range(WIN)
                    ]
                    for (i, j) in _NET8:
                        _ce(wb, i, j)
                    ts = [jnp.maximum(st[gg][i], wb[K - 1 - i]) for i in range(K)]
                    for (i, j) in _BITONIC8:
                        _ce(ts, i, j)
                    out_st.append(tuple(ts))
                return tuple(out_st)

            pair = (states[half * 2], states[half * 2 + 1])
            pair = lax.fori_loop(0, NWIN, body, pair)
            new_states.extend(pair)
        states = tuple(new_states)

    lane = lax.iota(jnp.int32, LANES)
    for g in range(NG):
        for j in range(K):
            idx = lane * K + (g * LANES * K + j)
            plsc.store_scatter(obuf, [idx], states[g][j])
    pltpu.sync_copy(obuf, out_hbm.at[b, pl.ds(ch0 * K, CH_HALF * K)])


def kernel(inputs):
    x = inputs.reshape(B, S, C)
    return _topk_sc(x)
